# Initial kernel scaffold; baseline (speedup 1.0000x reference)
import jax
import jax.numpy as jnp
from jax.experimental import pallas as pl


def kernel(inputs, seq_lens, weight, bias):
    raise NotImplementedError("write your pallas kernel here")


# speedup vs baseline: 1.2252x; 1.2252x over previous
import functools

import jax
import jax.numpy as jnp
from jax import lax
from jax.experimental import pallas as pl
from jax.experimental.pallas import tpu as pltpu

B, S, D = 16, 4096, 512
SB = 2048
S_BLKS = S // SB
EPS = 1e-5


def _stats_body(lens_ref, x_ref, sum_ref, sq_ref, cnt_ref):
    b = pl.program_id(0)
    j = pl.program_id(1)

    @pl.when(jnp.logical_and(b == 0, j == 0))
    def _init():
        sum_ref[...] = jnp.zeros_like(sum_ref)
        sq_ref[...] = jnp.zeros_like(sq_ref)
        cnt_ref[0, 0] = 0

    seq_len = lens_ref[b]
    rel = seq_len - j * SB
    x = x_ref[0]
    iota = lax.broadcasted_iota(jnp.int32, (SB, 1), 0)
    valid = iota < rel
    xm = jnp.where(valid, x, 0.0)
    sum_ref[0, :] += xm.sum(axis=0)
    sq_ref[0, :] += (xm * xm).sum(axis=0)
    cnt_ref[0, 0] += jnp.clip(rel, 0, SB)


def _norm_body(lens_ref, x_ref, sum_ref, sq_ref, cnt_ref, w_ref, b_ref, o_ref):
    b = pl.program_id(0)
    j = pl.program_id(1)
    cnt = jnp.maximum(cnt_ref[0, 0], 1).astype(jnp.float32)
    mean = sum_ref[0, :] / cnt
    var = jnp.maximum(sq_ref[0, :] / cnt - mean * mean, 0.0)
    scale = w_ref[0, :] * lax.rsqrt(var + EPS)
    shift = b_ref[0, :] - mean * scale

    seq_len = lens_ref[b]
    rel = seq_len - j * SB
    x = x_ref[0]
    iota = lax.broadcasted_iota(jnp.int32, (SB, 1), 0)
    valid = iota < rel
    o_ref[0] = jnp.where(valid, x * scale[None, :] + shift[None, :], 0.0)


@jax.jit
def _vlbn(x, lens32, weight, bias):
    grid = (B, S_BLKS)
    lens_spec = pl.BlockSpec(memory_space=pltpu.SMEM)
    x_spec = pl.BlockSpec((1, SB, D), lambda b, j: (b, j, 0))
    vec_spec = pl.BlockSpec((1, D), lambda b, j: (0, 0))
    cnt_spec = pl.BlockSpec((1, 1), lambda b, j: (0, 0), memory_space=pltpu.SMEM)

    s, sq, cnt = pl.pallas_call(
        _stats_body,
        grid=grid,
        in_specs=[lens_spec, x_spec],
        out_specs=[vec_spec, vec_spec, cnt_spec],
        out_shape=[
            jax.ShapeDtypeStruct((1, D), jnp.float32),
            jax.ShapeDtypeStruct((1, D), jnp.float32),
            jax.ShapeDtypeStruct((1, 1), jnp.int32),
        ],
    )(lens32, x)

    out = pl.pallas_call(
        _norm_body,
        grid=grid,
        in_specs=[lens_spec, x_spec, vec_spec, vec_spec, cnt_spec,
                  vec_spec, vec_spec],
        out_specs=x_spec,
        out_shape=jax.ShapeDtypeStruct((B, S, D), jnp.float32),
    )(lens32, x, s, sq, cnt, weight.reshape(1, D), bias.reshape(1, D))
    return out


def kernel(inputs, seq_lens, weight, bias):
    lens32 = seq_lens.astype(jnp.int32)
    with jax.enable_x64(False):
        return _vlbn(inputs.astype(jnp.float32), lens32,
                     weight.astype(jnp.float32), bias.astype(jnp.float32))
